# baseline (speedup 1.0000x reference)
import functools

import jax
import jax.numpy as jnp
from jax import lax
from jax.experimental import pallas as pl
from jax.experimental.pallas import tpu as pltpu, tpu_sc as plsc

_NP = 48
_NV = 10000
_F = 128
_STRIPE = 625
_SUB = 125
_PHASES = _STRIPE // _SUB
_PANELS_PER_CORE = _NP // 2


def _compute(xin, win, xout):
    def row(r, _):
        for l in range(_F // 16):
            sl = pl.ds(l * 16, 16)
            xout[r, sl] = xin[r, sl] + win[r, sl]
        return _
    lax.fori_loop(0, _SUB, row, None)


def _sc_body(xf_hbm, w_hbm, out_hbm, xin0, xin1, win, xo0, xo1,
             s_in0, s_in1, s_o0, s_o1):
    c = lax.axis_index("c")
    s = lax.axis_index("s")
    stripe0 = s * _STRIPE
    panel0 = c * _PANELS_PER_CORE

    for j in range(_PHASES):
        wrow0 = stripe0 + j * _SUB
        pltpu.sync_copy(w_hbm.at[pl.ds(wrow0, _SUB)], win)

        def row0(p):
            return (panel0 + p) * _NV + wrow0

        def start_in(p, buf, sem):
            pltpu.make_async_copy(xf_hbm.at[pl.ds(row0(p), _SUB)], buf, sem).start()

        def wait_in(buf, sem):
            pltpu.make_async_copy(xf_hbm.at[pl.ds(0, _SUB)], buf, sem).wait()

        def start_out(p, buf, sem):
            pltpu.make_async_copy(buf, out_hbm.at[pl.ds(row0(p), _SUB)], sem).start()

        def wait_out(buf, sem):
            pltpu.make_async_copy(buf, out_hbm.at[pl.ds(0, _SUB)], sem).wait()

        start_in(0, xin0, s_in0)

        def step(o, _):
            p0 = 2 * o
            p1 = p0 + 1
            wait_in(xin0, s_in0)
            start_in(p1, xin1, s_in1)

            @pl.when(o > 0)
            def _():
                wait_out(xo0, s_o0)

            _compute(xin0, win, xo0)
            start_out(p0, xo0, s_o0)

            @pl.when(o < _PANELS_PER_CORE // 2 - 1)
            def _():
                start_in(p0 + 2, xin0, s_in0)

            wait_in(xin1, s_in1)

            @pl.when(o > 0)
            def _():
                wait_out(xo1, s_o1)

            _compute(xin1, win, xo1)
            start_out(p1, xo1, s_o1)
            return _

        lax.fori_loop(0, _PANELS_PER_CORE // 2, step, None)
        wait_out(xo0, s_o0)
        wait_out(xo1, s_o1)


def kernel(x, W):
    batch, n, t, f = x.shape
    xt = jnp.transpose(x, (0, 2, 1, 3))
    xf = xt.reshape(batch * t * n, f)

    mesh = plsc.VectorSubcoreMesh(core_axis_name="c", subcore_axis_name="s")
    sc = functools.partial(
        pl.kernel,
        out_type=jax.ShapeDtypeStruct((batch * t * n, f), x.dtype),
        mesh=mesh,
        compiler_params=pltpu.CompilerParams(use_tc_tiling_on_sc=False),
        scratch_types=[
            pltpu.VMEM((_SUB, _F), jnp.float32),
            pltpu.VMEM((_SUB, _F), jnp.float32),
            pltpu.VMEM((_SUB, _F), jnp.float32),
            pltpu.VMEM((_SUB, _F), jnp.float32),
            pltpu.VMEM((_SUB, _F), jnp.float32),
            pltpu.SemaphoreType.DMA,
            pltpu.SemaphoreType.DMA,
            pltpu.SemaphoreType.DMA,
            pltpu.SemaphoreType.DMA,
        ],
    )(_sc_body)
    out_f = sc(xf, W)
    out_t = out_f.reshape(batch, t, n, f)
    return jnp.transpose(out_t, (0, 2, 1, 3))


# speedup vs baseline: 1.4337x; 1.4337x over previous
import jax
import jax.numpy as jnp
from jax.experimental import pallas as pl


def _add_kernel(x_ref, w_ref, o_ref):
    o_ref[...] = x_ref[...] + w_ref[...][None, None, :, :]


def kernel(x, W):
    batch, n, t, f = x.shape
    xt = jnp.transpose(x, (0, 2, 1, 3))
    nb = 10000
    ts = 2
    out_t = pl.pallas_call(
        _add_kernel,
        grid=(n // nb, batch, t // ts),
        in_specs=[
            pl.BlockSpec((1, ts, nb, f), lambda i, b, s: (b, s, i, 0)),
            pl.BlockSpec((nb, f), lambda i, b, s: (i, 0)),
        ],
        out_specs=pl.BlockSpec((1, ts, nb, f), lambda i, b, s: (b, s, i, 0)),
        out_shape=jax.ShapeDtypeStruct((batch, t, n, f), x.dtype),
    )(xt, W)
    return jnp.transpose(out_t, (0, 2, 1, 3))
